# baseline (speedup 1.0000x reference)
import jax
import jax.numpy as jnp
from jax.experimental import pallas as pl
from jax.experimental.pallas import tpu as pltpu

_NUM_EXPERTS = 16
_TOP_K = 8
_HIDDEN = 768
_INNER = _HIDDEN * 4


def _moe_kernel(x_ref, rw_ref, rb_ref, pw_hbm, pb_ref, ow_hbm, ob_ref,
                out_ref, logits_ref,
                pbuf, obuf, comb_ref, psem, osem):
    e = pl.program_id(0)
    E = _NUM_EXPERTS
    H = _HIDDEN
    I = _INNER
    HH = H // 2
    HQ = H // 4

    def proj_copies(u):
        slot = jax.lax.rem(u, 3)
        ex = jax.lax.div(u, 2)
        half = jax.lax.rem(u, 2)
        r0 = half * HH
        c0 = pltpu.make_async_copy(
            pw_hbm.at[ex, pl.ds(r0, HQ), :],
            pbuf.at[slot, pl.ds(0, HQ), :], psem.at[slot, 0])
        c1 = pltpu.make_async_copy(
            pw_hbm.at[ex, pl.ds(r0 + HQ, HQ), :],
            pbuf.at[slot, pl.ds(HQ, HQ), :], psem.at[slot, 1])
        return c0, c1

    def out_copies(ex):
        slot = jax.lax.rem(ex, 2)
        c0 = pltpu.make_async_copy(
            ow_hbm.at[ex, pl.ds(0, I // 2), :],
            obuf.at[slot, pl.ds(0, I // 2), :], osem.at[slot, 0])
        c1 = pltpu.make_async_copy(
            ow_hbm.at[ex, pl.ds(I // 2, I // 2), :],
            obuf.at[slot, pl.ds(I // 2, I // 2), :], osem.at[slot, 1])
        return c0, c1

    def start_proj(u):
        for cp in proj_copies(u):
            cp.start()

    def start_out(ex):
        for cp in out_copies(ex):
            cp.start()

    @pl.when(e == 0)
    def _prologue():
        start_proj(0)
        start_proj(1)
        start_out(0)
        start_proj(2)
        start_out(1)
        x = x_ref[...]
        logits = jnp.dot(x, rw_ref[...], preferred_element_type=jnp.float32)
        logits = logits + rb_ref[...]
        logits_ref[...] = logits
        m = jnp.max(logits, axis=-1, keepdims=True)
        ex_ = jnp.exp(logits - m)
        probs = ex_ / jnp.sum(ex_, axis=-1, keepdims=True)
        t, ne = probs.shape
        col = jax.lax.broadcasted_iota(jnp.int32, (t, ne), 1)
        rank = jnp.zeros((t, ne), jnp.int32)
        for j in range(ne):
            pj = probs[:, j:j + 1]
            beats = (pj > probs) | ((pj == probs) & (j < col))
            rank = rank + beats.astype(jnp.int32)
        comb_ref[...] = jnp.where(rank < _TOP_K, probs, 0.0)
        out_ref[...] = jnp.zeros_like(out_ref)

    for cp in proj_copies(2 * e):
        cp.wait()
    for cp in proj_copies(2 * e + 1):
        cp.wait()
    for cp in out_copies(e):
        cp.wait()

    slot0 = jax.lax.rem(2 * e, 3)
    slot1 = jax.lax.rem(2 * e + 1, 3)
    x = x_ref[...]
    x_lo = x[:, :HH]
    x_hi = x[:, HH:]
    p0 = pbuf[slot0]
    p1 = pbuf[slot1]
    ow = obuf[jax.lax.rem(e, 2)]
    pb = pb_ref[0, 0, :]
    comb = comb_ref[...]
    lane = jax.lax.broadcasted_iota(jnp.int32, comb.shape, 1)
    w = jnp.sum(jnp.where(lane == e, comb, 0.0), axis=1)[:, None]

    C = I // 2
    for c in range(2):
        ca = slice(c * C, (c + 1) * C)
        cg = slice(I + c * C, I + (c + 1) * C)
        a = (jnp.dot(x_lo, p0[:, ca], preferred_element_type=jnp.float32)
             + jnp.dot(x_hi, p1[:, ca], preferred_element_type=jnp.float32)
             + pb[ca][None, :])
        g = (jnp.dot(x_lo, p0[:, cg], preferred_element_type=jnp.float32)
             + jnp.dot(x_hi, p1[:, cg], preferred_element_type=jnp.float32)
             + pb[cg][None, :])
        act = a * (g * jax.nn.sigmoid(g))
        y = jnp.dot(act, ow[ca, :], preferred_element_type=jnp.float32)
        out_ref[...] += y * w

    out_ref[...] += w * ob_ref[0, 0, :][None, :]

    @pl.when(2 * e + 3 < 2 * E)
    def _refill_p0():
        start_proj(2 * e + 3)

    @pl.when(2 * e + 4 < 2 * E)
    def _refill_p1():
        start_proj(2 * e + 4)

    @pl.when(e + 2 < E)
    def _refill_o():
        start_out(e + 2)


def kernel(hidden_states, router_W, router_b, proj_W, proj_b, out_W, out_b):
    B, S, H = hidden_states.shape
    T = B * S
    E = _NUM_EXPERTS
    I = _INNER

    x = hidden_states.reshape(T, H)
    proj_b3 = proj_b.reshape(E, 1, 2 * I)
    out_b3 = out_b.reshape(E, 1, H)

    out, logits = pl.pallas_call(
        _moe_kernel,
        grid=(E,),
        in_specs=[
            pl.BlockSpec((T, H), lambda e: (0, 0)),
            pl.BlockSpec((H, E), lambda e: (0, 0)),
            pl.BlockSpec((1, E), lambda e: (0, 0)),
            pl.BlockSpec(memory_space=pl.ANY),
            pl.BlockSpec((1, 1, 2 * I), lambda e: (e, 0, 0)),
            pl.BlockSpec(memory_space=pl.ANY),
            pl.BlockSpec((1, 1, H), lambda e: (e, 0, 0)),
        ],
        out_specs=(
            pl.BlockSpec((T, H), lambda e: (0, 0)),
            pl.BlockSpec((T, E), lambda e: (0, 0)),
        ),
        out_shape=(
            jax.ShapeDtypeStruct((T, H), jnp.float32),
            jax.ShapeDtypeStruct((T, E), jnp.float32),
        ),
        scratch_shapes=[
            pltpu.VMEM((3, H // 2, 2 * I), jnp.float32),
            pltpu.VMEM((2, I, H), jnp.float32),
            pltpu.VMEM((T, E), jnp.float32),
            pltpu.SemaphoreType.DMA((3, 2)),
            pltpu.SemaphoreType.DMA((2, 2)),
        ],
        compiler_params=pltpu.CompilerParams(
            dimension_semantics=("arbitrary",),
        ),
    )(x, router_W, router_b.reshape(1, E), proj_W, proj_b3, out_W, out_b3)

    return out.reshape(B, S, H), logits.reshape(B, S, E)


# speedup vs baseline: 1.0119x; 1.0119x over previous
import jax
import jax.numpy as jnp
from jax.experimental import pallas as pl
from jax.experimental.pallas import tpu as pltpu

_NUM_EXPERTS = 16
_TOP_K = 8
_HIDDEN = 768
_INNER = _HIDDEN * 4
_CHUNK = 1024


def _moe_kernel(x_ref, rw_ref, rb_ref, pa_lo_ref, pa_hi_ref, pg_lo_ref,
                pg_hi_ref, pba_ref, pbg_ref, ow_lo_ref, ow_hi_ref, ob_ref,
                out_ref, logits_ref, comb_ref):
    e = pl.program_id(0)
    c = pl.program_id(1)

    @pl.when((e == 0) & (c == 0))
    def _router_and_init():
        x = x_ref[...]
        logits = jnp.dot(x, rw_ref[...], preferred_element_type=jnp.float32)
        logits = logits + rb_ref[...]
        logits_ref[...] = logits
        m = jnp.max(logits, axis=-1, keepdims=True)
        ex = jnp.exp(logits - m)
        probs = ex / jnp.sum(ex, axis=-1, keepdims=True)
        t, ne = probs.shape
        col = jax.lax.broadcasted_iota(jnp.int32, (t, ne), 1)
        rank = jnp.zeros((t, ne), jnp.int32)
        for j in range(ne):
            pj = probs[:, j:j + 1]
            beats = (pj > probs) | ((pj == probs) & (j < col))
            rank = rank + beats.astype(jnp.int32)
        comb_ref[...] = jnp.where(rank < _TOP_K, probs, 0.0)
        out_ref[...] = jnp.zeros_like(out_ref)

    x = x_ref[...]
    HH = _HIDDEN // 2
    C = _CHUNK
    x_lo = x[:, :HH]
    x_hi = x[:, HH:]
    a = (jnp.dot(x_lo, pa_lo_ref[0], preferred_element_type=jnp.float32)
         + jnp.dot(x_hi, pa_hi_ref[0], preferred_element_type=jnp.float32)
         + pba_ref[0, 0, :][None, :])
    g = (jnp.dot(x_lo, pg_lo_ref[0], preferred_element_type=jnp.float32)
         + jnp.dot(x_hi, pg_hi_ref[0], preferred_element_type=jnp.float32)
         + pbg_ref[0, 0, :][None, :])
    act = a * (g * jax.nn.sigmoid(g))
    y = (jnp.dot(act[:, :C // 2], ow_lo_ref[0],
                 preferred_element_type=jnp.float32)
         + jnp.dot(act[:, C // 2:], ow_hi_ref[0],
                   preferred_element_type=jnp.float32))

    comb = comb_ref[...]
    lane = jax.lax.broadcasted_iota(jnp.int32, comb.shape, 1)
    w = jnp.sum(jnp.where(lane == e, comb, 0.0), axis=1)[:, None]
    contrib = y * w

    @pl.when(c == 0)
    def _bias():
        out_ref[...] += w * ob_ref[0, 0, :][None, :]

    out_ref[...] += contrib


def kernel(hidden_states, router_W, router_b, proj_W, proj_b, out_W, out_b):
    B, S, H = hidden_states.shape
    T = B * S
    E = _NUM_EXPERTS
    I = _INNER
    C = _CHUNK
    NC = I // C

    x = hidden_states.reshape(T, H)
    proj_b3 = proj_b.reshape(E, 1, 2 * I)
    out_b3 = out_b.reshape(E, 1, H)

    out, logits = pl.pallas_call(
        _moe_kernel,
        grid=(E, NC),
        in_specs=[
            pl.BlockSpec((T, H), lambda e, c: (0, 0)),
            pl.BlockSpec((H, E), lambda e, c: (0, 0)),
            pl.BlockSpec((1, E), lambda e, c: (0, 0)),
            pl.BlockSpec((1, H // 2, C), lambda e, c: (e, 0, c)),
            pl.BlockSpec((1, H // 2, C), lambda e, c: (e, 1, c)),
            pl.BlockSpec((1, H // 2, C), lambda e, c: (e, 0, c + NC)),
            pl.BlockSpec((1, H // 2, C), lambda e, c: (e, 1, c + NC)),
            pl.BlockSpec((1, 1, C), lambda e, c: (e, 0, c)),
            pl.BlockSpec((1, 1, C), lambda e, c: (e, 0, c + NC)),
            pl.BlockSpec((1, C // 2, H), lambda e, c: (e, 2 * c, 0)),
            pl.BlockSpec((1, C // 2, H), lambda e, c: (e, 2 * c + 1, 0)),
            pl.BlockSpec((1, 1, H), lambda e, c: (e, 0, 0)),
        ],
        out_specs=(
            pl.BlockSpec((T, H), lambda e, c: (0, 0)),
            pl.BlockSpec((T, E), lambda e, c: (0, 0)),
        ),
        out_shape=(
            jax.ShapeDtypeStruct((T, H), jnp.float32),
            jax.ShapeDtypeStruct((T, E), jnp.float32),
        ),
        scratch_shapes=[
            pltpu.VMEM((T, E), jnp.float32),
        ],
        compiler_params=pltpu.CompilerParams(
            dimension_semantics=("arbitrary", "arbitrary"),
        ),
    )(x, router_W, router_b.reshape(1, E), proj_W, proj_W, proj_W, proj_W,
      proj_b3, proj_b3, out_W, out_W, out_b3)

    return out.reshape(B, S, H), logits.reshape(B, S, E)
